# Initial kernel scaffold; baseline (speedup 1.0000x reference)
import jax
import jax.numpy as jnp
from jax.experimental import pallas as pl


def kernel(feature_indices, feature_values, weight, bias):
    raise NotImplementedError("write your pallas kernel here")


# speedup vs baseline: 1.8444x; 1.8444x over previous
import functools

import jax
import jax.numpy as jnp
import numpy as np
from jax import lax
from jax.experimental import pallas as pl
from jax.experimental.pallas import tpu as pltpu
from jax.experimental.pallas import tpu_sc as plsc

_B = 16384
_K = 50
_D = 128
_L = 16
_NC = 2
_NS = 16
_NW = _NC * _NS
_BPW = _B // _NW
_CHUNK = 128
_NCHUNK = _BPW // _CHUNK
_NDC = _D // _L
_KP = 64

_BCAST_DNUMS = lax.GatherDimensionNumbers(
    offset_dims=(), collapsed_slice_dims=(0,), start_index_map=(0,))


def _bcast_lane(vec, lane):
    idx = jnp.full((_L,), lane, jnp.int32).reshape(_L, 1)
    return lax.gather(vec, idx, _BCAST_DNUMS, (1,),
                      mode=lax.GatherScatterMode.PROMISE_IN_BOUNDS)


def _tec_body(idx_hbm, val_hbm, w_hbm, bias_hbm, out_hbm,
              idx_v, val_v, rows_v, out_v, bias_v, gsem):
    cidx = lax.axis_index("c")
    sidx = lax.axis_index("s")
    wid = sidx * _NC + cidx
    base = wid * _BPW

    pltpu.sync_copy(bias_hbm, bias_v)
    bias_vecs = tuple(bias_v[pl.ds(c * _L, _L)] for c in range(_NDC))

    for chunk in range(_NCHUNK):
        cbase = base + chunk * _CHUNK
        pltpu.sync_copy(idx_hbm.at[pl.ds(cbase, _CHUNK)], idx_v)
        pltpu.sync_copy(val_hbm.at[pl.ds(cbase, _CHUNK)], val_v)

        def row(i, _):
            pltpu.async_copy(w_hbm.at[idx_v.at[i]], rows_v, gsem).wait()
            vrows = [val_v[i, pl.ds(g * _L, _L)] for g in range(_KP // _L)]
            acc = list(bias_vecs)
            for k in range(_K):
                g, lane = divmod(k, _L)
                bc = _bcast_lane(vrows[g], lane)
                for c in range(_NDC):
                    acc[c] = acc[c] + rows_v[k, pl.ds(c * _L, _L)] * bc
            for c in range(_NDC):
                out_v[i, pl.ds(c * _L, _L)] = acc[c]
            return 0

        lax.fori_loop(0, _CHUNK, row, 0)
        pltpu.sync_copy(out_v, out_hbm.at[pl.ds(cbase, _CHUNK)])


@functools.partial(
    pl.kernel,
    out_type=jax.ShapeDtypeStruct((_B, _D), jnp.float32),
    mesh=plsc.VectorSubcoreMesh(core_axis_name="c", subcore_axis_name="s"),
    scratch_types=[
        pltpu.VMEM((_CHUNK, _K), jnp.int32),
        pltpu.VMEM((_CHUNK, _KP), jnp.float32),
        pltpu.VMEM((_K, _D), jnp.float32),
        pltpu.VMEM((_CHUNK, _D), jnp.float32),
        pltpu.VMEM((_D,), jnp.float32),
        pltpu.SemaphoreType.DMA,
    ],
)
def _ft_sc(idx_hbm, val_hbm, w_hbm, bias_hbm, out_hbm,
           idx_v, val_v, rows_v, out_v, bias_v, gsem):
    _tec_body(idx_hbm, val_hbm, w_hbm, bias_hbm, out_hbm,
              idx_v, val_v, rows_v, out_v, bias_v, gsem)


def kernel(feature_indices, feature_values, weight, bias):
    vals_padded = jnp.pad(feature_values, ((0, 0), (0, _KP - _K)))
    return _ft_sc(feature_indices, vals_padded, weight, bias)


# speedup vs baseline: 3.0186x; 1.6367x over previous
import functools

import jax
import jax.numpy as jnp
import numpy as np
from jax import lax
from jax.experimental import pallas as pl
from jax.experimental.pallas import tpu as pltpu
from jax.experimental.pallas import tpu_sc as plsc

_B = 16384
_K = 50
_D = 128
_L = 16
_NC = 2
_NS = 16
_NW = _NC * _NS
_BPW = _B // _NW
_CHUNK = 128
_NCHUNK = _BPW // _CHUNK
_NDC = _D // _L
_KP = 64
_NBUF = 4

_BCAST_DNUMS = lax.GatherDimensionNumbers(
    offset_dims=(), collapsed_slice_dims=(0,), start_index_map=(0,))


def _bcast_lane(vec, lane):
    idx = jnp.full((_L,), lane, jnp.int32).reshape(_L, 1)
    return lax.gather(vec, idx, _BCAST_DNUMS, (1,),
                      mode=lax.GatherScatterMode.PROMISE_IN_BOUNDS)


def _tec_body(idx_hbm, val_hbm, w_hbm, bias_hbm, out_hbm,
              idx_v, val_v, rows_v, out_v, bias_v, sems):
    cidx = lax.axis_index("c")
    sidx = lax.axis_index("s")
    wid = sidx * _NC + cidx
    base = wid * _BPW

    pltpu.sync_copy(bias_hbm, bias_v)
    bias_vecs = tuple(bias_v[pl.ds(c * _L, _L)] for c in range(_NDC))

    def compute_row(i, rbuf):
        vrows = [val_v[i, pl.ds(g * _L, _L)] for g in range(_KP // _L)]
        acc = list(bias_vecs)
        for k in range(_K):
            g, lane = divmod(k, _L)
            bc = _bcast_lane(vrows[g], lane)
            for c in range(_NDC):
                acc[c] = acc[c] + rbuf[k, pl.ds(c * _L, _L)] * bc
        for c in range(_NDC):
            out_v[i, pl.ds(c * _L, _L)] = acc[c]

    def chunk_body(chunk, _):
        cbase = base + chunk * _CHUNK
        pltpu.sync_copy(idx_hbm.at[pl.ds(cbase, _CHUNK)], idx_v)
        pltpu.sync_copy(val_hbm.at[pl.ds(cbase, _CHUNK)], val_v)

        for b in range(_NBUF):
            pltpu.async_copy(w_hbm.at[idx_v.at[b]], rows_v.at[b], sems[b])

        def ring_round(r, _):
            i0 = r * _NBUF
            for b in range(_NBUF):
                i = i0 + b
                pltpu.make_async_copy(
                    w_hbm.at[idx_v.at[i]], rows_v.at[b], sems[b]).wait()
                compute_row(i, rows_v.at[b])
                inext = i + _NBUF

                @pl.when(inext < _CHUNK)
                def _():
                    pltpu.async_copy(
                        w_hbm.at[idx_v.at[inext]], rows_v.at[b], sems[b])
            return 0

        lax.fori_loop(0, _CHUNK // _NBUF, ring_round, 0)
        pltpu.sync_copy(out_v, out_hbm.at[pl.ds(cbase, _CHUNK)])
        return 0

    lax.fori_loop(0, _NCHUNK, chunk_body, 0)


@functools.partial(
    pl.kernel,
    out_type=jax.ShapeDtypeStruct((_B, _D), jnp.float32),
    mesh=plsc.VectorSubcoreMesh(core_axis_name="c", subcore_axis_name="s"),
    scratch_types=[
        pltpu.VMEM((_CHUNK, _K), jnp.int32),
        pltpu.VMEM((_CHUNK, _KP), jnp.float32),
        pltpu.VMEM((_NBUF, _K, _D), jnp.float32),
        pltpu.VMEM((_CHUNK, _D), jnp.float32),
        pltpu.VMEM((_D,), jnp.float32),
    ] + [pltpu.SemaphoreType.DMA] * _NBUF,
)
def _ft_sc(idx_hbm, val_hbm, w_hbm, bias_hbm, out_hbm,
           idx_v, val_v, rows_v, out_v, bias_v, *sems):
    _tec_body(idx_hbm, val_hbm, w_hbm, bias_hbm, out_hbm,
              idx_v, val_v, rows_v, out_v, bias_v, sems)


def kernel(feature_indices, feature_values, weight, bias):
    vals_padded = jnp.pad(feature_values, ((0, 0), (0, _KP - _K)))
    return _ft_sc(feature_indices, vals_padded, weight, bias)
